# baseline (speedup 1.0000x reference)
import jax
import jax.numpy as jnp
from jax.experimental import pallas as pl
from jax.experimental.pallas import tpu as pltpu

N_NODES = 50000
IN_CH = 256
NUM_HEADS = 4
OUT_CH = 256
OUT_PER_HEAD = OUT_CH // NUM_HEADS
NUM_GRAPHS = 64

BLK = 5000
NB = N_NODES // BLK


def _agg_kernel(x_ref, batch_ref, gate_w_ref, nn_w_ref, nn_b_ref, out_ref,
                m_ref, d_ref, pooled_ref):
    i = pl.program_id(0)

    xb = x_ref[...]
    bt = batch_ref[0]
    gidx = jax.lax.broadcasted_iota(jnp.int32, (NUM_GRAPHS, BLK), 0)
    maskf = (bt == gidx).astype(jnp.float32)
    mask16 = maskf.astype(jnp.bfloat16)
    xb16 = xb.astype(jnp.bfloat16)

    gateT = jax.lax.dot_general(
        gate_w_ref[...], xb, (((1,), (1,)), ((), ())),
        preferred_element_type=jnp.float32)

    @pl.when(i == 0)
    def _init():
        m_ref[...] = jnp.max(gateT, axis=1, keepdims=True)
        d_ref[...] = jnp.zeros((NUM_GRAPHS, NUM_HEADS), jnp.float32)
        pooled_ref[...] = jnp.zeros((NUM_HEADS * NUM_GRAPHS, IN_CH), jnp.float32)

    basis = m_ref[...]
    en = jnp.exp(gateT - basis)
    en16 = en.astype(jnp.bfloat16)

    bd = jax.lax.dot_general(
        mask16, en16, (((1,), (1,)), ((), ())),
        preferred_element_type=jnp.float32)
    d_ref[...] += bd

    e_rows = [mask16 * en16[h:h + 1, :] for h in range(NUM_HEADS)]
    et = jnp.concatenate(e_rows, axis=0)
    pooled_ref[...] += jnp.dot(et, xb16, preferred_element_type=jnp.float32)

    @pl.when(i == NB - 1)
    def _finalize():
        for h in range(NUM_HEADS):
            dh = d_ref[:, h:h + 1]
            safe = jnp.where(dh > 0.0, dh, 1.0)
            sl = slice(h * NUM_GRAPHS, (h + 1) * NUM_GRAPHS)
            ph = pooled_ref[sl, :] / safe
            oh = jax.lax.dot_general(
                ph, nn_w_ref[h], (((1,), (1,)), ((), ())),
                preferred_element_type=jnp.float32)
            oh = oh + jnp.where(dh > 0.0, 1.0, 0.0) * nn_b_ref[h:h + 1, :]
            out_ref[:, h * OUT_PER_HEAD:(h + 1) * OUT_PER_HEAD] = oh


def kernel(x, batch, gate_w, nn_w, nn_b):
    batch3d = batch.astype(jnp.int32).reshape(NB, 1, BLK)
    nn_b2 = nn_b.reshape(NUM_HEADS, OUT_PER_HEAD)
    return pl.pallas_call(
        _agg_kernel,
        grid=(NB,),
        in_specs=[
            pl.BlockSpec((BLK, IN_CH), lambda i: (i, 0)),
            pl.BlockSpec((1, 1, BLK), lambda i: (i, 0, 0)),
            pl.BlockSpec((NUM_HEADS, IN_CH), lambda i: (0, 0)),
            pl.BlockSpec((NUM_HEADS, OUT_PER_HEAD, IN_CH), lambda i: (0, 0, 0)),
            pl.BlockSpec((NUM_HEADS, OUT_PER_HEAD), lambda i: (0, 0)),
        ],
        out_specs=pl.BlockSpec((NUM_GRAPHS, OUT_CH), lambda i: (0, 0)),
        out_shape=jax.ShapeDtypeStruct((NUM_GRAPHS, OUT_CH), jnp.float32),
        scratch_shapes=[
            pltpu.VMEM((NUM_HEADS, 1), jnp.float32),
            pltpu.VMEM((NUM_GRAPHS, NUM_HEADS), jnp.float32),
            pltpu.VMEM((NUM_HEADS * NUM_GRAPHS, IN_CH), jnp.float32),
        ],
    )(x, batch3d, gate_w, nn_w, nn_b2)


# speedup vs baseline: 1.0742x; 1.0742x over previous
import jax
import jax.numpy as jnp
from jax.experimental import pallas as pl
from jax.experimental.pallas import tpu as pltpu

N_NODES = 50000
IN_CH = 256
NUM_HEADS = 4
OUT_CH = 256
OUT_PER_HEAD = OUT_CH // NUM_HEADS
NUM_GRAPHS = 64

BLK = 5000
NB = N_NODES // BLK
_CHUNKS = [(0, 1280), (1280, 1280), (2560, 1280), (3840, 1160)]


def _agg_kernel(x_ref, batch_ref, gate_w_ref, nn_w_ref, nn_b_ref, out_ref,
                m_ref, d_ref, pooled_ref):
    i = pl.program_id(0)

    xb = x_ref[...]
    xb16 = xb.astype(jnp.bfloat16)
    bt = batch_ref[0]
    gidx = jax.lax.broadcasted_iota(jnp.int16, (NUM_GRAPHS, BLK), 0)
    mask = bt == gidx

    gateT = jax.lax.dot_general(
        gate_w_ref[...], xb16, (((1,), (1,)), ((), ())),
        preferred_element_type=jnp.float32)

    @pl.when(i == 0)
    def _init():
        m_ref[...] = jnp.max(gateT, axis=1, keepdims=True)
        d_ref[...] = jnp.zeros((NUM_GRAPHS, NUM_HEADS), jnp.float32)
        pooled_ref[...] = jnp.zeros((NUM_HEADS * NUM_GRAPHS, IN_CH), jnp.float32)

    basis = m_ref[...]
    en16 = jnp.exp(gateT - basis).astype(jnp.bfloat16)

    zero16 = jnp.zeros((), jnp.bfloat16)
    one16 = jnp.ones((), jnp.bfloat16)

    for s, w in _CHUNKS:
        mask_c = mask[:, s:s + w]
        en16_c = en16[:, s:s + w]
        xb16_c = xb16[s:s + w, :]
        mask16_c = jnp.where(mask_c, one16, zero16)
        bd = jax.lax.dot_general(
            mask16_c, en16_c, (((1,), (1,)), ((), ())),
            preferred_element_type=jnp.float32)
        d_ref[...] += bd
        e_rows = [jnp.where(mask_c, en16_c[h:h + 1, :], zero16)
                  for h in range(NUM_HEADS)]
        et = jnp.concatenate(e_rows, axis=0)
        pooled_ref[...] += jnp.dot(et, xb16_c,
                                   preferred_element_type=jnp.float32)

    @pl.when(i == NB - 1)
    def _finalize():
        for h in range(NUM_HEADS):
            sl = slice(h * NUM_GRAPHS, (h + 1) * NUM_GRAPHS)
            dh = d_ref[:, h:h + 1]
            safe = jnp.where(dh > 0.0, dh, 1.0)
            ph = pooled_ref[sl, :] / safe
            oh = jax.lax.dot_general(
                ph, nn_w_ref[h], (((1,), (1,)), ((), ())),
                preferred_element_type=jnp.float32)
            oh = oh + jnp.where(dh > 0.0, 1.0, 0.0) * nn_b_ref[h:h + 1, :]
            out_ref[:, h * OUT_PER_HEAD:(h + 1) * OUT_PER_HEAD] = oh


def kernel(x, batch, gate_w, nn_w, nn_b):
    batch3d = batch.astype(jnp.int16).reshape(NB, 1, BLK)
    nn_b2 = nn_b.reshape(NUM_HEADS, OUT_PER_HEAD)
    return pl.pallas_call(
        _agg_kernel,
        grid=(NB,),
        in_specs=[
            pl.BlockSpec((BLK, IN_CH), lambda i: (i, 0)),
            pl.BlockSpec((1, 1, BLK), lambda i: (i, 0, 0)),
            pl.BlockSpec((NUM_HEADS, IN_CH), lambda i: (0, 0)),
            pl.BlockSpec((NUM_HEADS, OUT_PER_HEAD, IN_CH), lambda i: (0, 0, 0)),
            pl.BlockSpec((NUM_HEADS, OUT_PER_HEAD), lambda i: (0, 0)),
        ],
        out_specs=pl.BlockSpec((NUM_GRAPHS, OUT_CH), lambda i: (0, 0)),
        out_shape=jax.ShapeDtypeStruct((NUM_GRAPHS, OUT_CH), jnp.float32),
        scratch_shapes=[
            pltpu.VMEM((NUM_HEADS, 1), jnp.float32),
            pltpu.VMEM((NUM_GRAPHS, NUM_HEADS), jnp.float32),
            pltpu.VMEM((NUM_HEADS * NUM_GRAPHS, IN_CH), jnp.float32),
        ],
    )(x, batch3d, gate_w, nn_w, nn_b2)


# speedup vs baseline: 1.2980x; 1.2083x over previous
import jax
import jax.numpy as jnp
from jax.experimental import pallas as pl
from jax.experimental.pallas import tpu as pltpu

N_NODES = 50000
IN_CH = 256
NUM_HEADS = 4
OUT_CH = 256
OUT_PER_HEAD = OUT_CH // NUM_HEADS
NUM_GRAPHS = 64

BLK = 5000
NB = N_NODES // BLK


def _agg_kernel(x_ref, batch_ref, gate_w_ref, nn_w_ref, nn_b_ref, out_ref,
                m_ref, d_ref, pooled_ref, et_ref, xs_ref):
    i = pl.program_id(0)
    slot = jax.lax.rem(i, 2)
    prev = jax.lax.rem(i + 1, 2)

    @pl.when(i < NB)
    def _build():
        xb = x_ref[...]
        xb16 = xb.astype(jnp.bfloat16)
        xs_ref[slot] = xb16
        bt = batch_ref[0]
        gidx = jax.lax.broadcasted_iota(jnp.int16, (NUM_GRAPHS, BLK), 0)
        mask = bt == gidx

        gateT = jax.lax.dot_general(
            gate_w_ref[...], xb16, (((1,), (1,)), ((), ())),
            preferred_element_type=jnp.float32)

        @pl.when(i == 0)
        def _init():
            m_ref[...] = jnp.max(gateT, axis=1, keepdims=True)
            d_ref[...] = jnp.zeros((NUM_GRAPHS, NUM_HEADS), jnp.float32)
            pooled_ref[...] = jnp.zeros((NUM_HEADS * NUM_GRAPHS, IN_CH),
                                        jnp.float32)

        basis = m_ref[...]
        en16 = jnp.exp(gateT - basis).astype(jnp.bfloat16)

        zero16 = jnp.zeros((), jnp.bfloat16)
        one16 = jnp.ones((), jnp.bfloat16)
        mask16 = jnp.where(mask, one16, zero16)
        bd = jax.lax.dot_general(
            mask16, en16, (((1,), (1,)), ((), ())),
            preferred_element_type=jnp.float32)
        d_ref[...] += bd

        e_rows = [jnp.where(mask, en16[h:h + 1, :], zero16)
                  for h in range(NUM_HEADS)]
        et_ref[slot] = jnp.concatenate(e_rows, axis=0)

    @pl.when(i > 0)
    def _matmul():
        pooled_ref[...] += jnp.dot(et_ref[prev], xs_ref[prev],
                                   preferred_element_type=jnp.float32)

    @pl.when(i == NB)
    def _finalize():
        for h in range(NUM_HEADS):
            sl = slice(h * NUM_GRAPHS, (h + 1) * NUM_GRAPHS)
            dh = d_ref[:, h:h + 1]
            safe = jnp.where(dh > 0.0, dh, 1.0)
            ph = pooled_ref[sl, :] / safe
            oh = jax.lax.dot_general(
                ph, nn_w_ref[h], (((1,), (1,)), ((), ())),
                preferred_element_type=jnp.float32)
            oh = oh + jnp.where(dh > 0.0, 1.0, 0.0) * nn_b_ref[h:h + 1, :]
            out_ref[:, h * OUT_PER_HEAD:(h + 1) * OUT_PER_HEAD] = oh


def kernel(x, batch, gate_w, nn_w, nn_b):
    batch3d = batch.astype(jnp.int16).reshape(NB, 1, BLK)
    nn_b2 = nn_b.reshape(NUM_HEADS, OUT_PER_HEAD)
    last = NB - 1
    return pl.pallas_call(
        _agg_kernel,
        grid=(NB + 1,),
        in_specs=[
            pl.BlockSpec((BLK, IN_CH), lambda i: (jnp.minimum(i, last), 0)),
            pl.BlockSpec((1, 1, BLK), lambda i: (jnp.minimum(i, last), 0, 0)),
            pl.BlockSpec((NUM_HEADS, IN_CH), lambda i: (0, 0)),
            pl.BlockSpec((NUM_HEADS, OUT_PER_HEAD, IN_CH), lambda i: (0, 0, 0)),
            pl.BlockSpec((NUM_HEADS, OUT_PER_HEAD), lambda i: (0, 0)),
        ],
        out_specs=pl.BlockSpec((NUM_GRAPHS, OUT_CH), lambda i: (0, 0)),
        out_shape=jax.ShapeDtypeStruct((NUM_GRAPHS, OUT_CH), jnp.float32),
        scratch_shapes=[
            pltpu.VMEM((NUM_HEADS, 1), jnp.float32),
            pltpu.VMEM((NUM_GRAPHS, NUM_HEADS), jnp.float32),
            pltpu.VMEM((NUM_HEADS * NUM_GRAPHS, IN_CH), jnp.float32),
            pltpu.VMEM((2, NUM_HEADS * NUM_GRAPHS, BLK), jnp.bfloat16),
            pltpu.VMEM((2, BLK, IN_CH), jnp.bfloat16),
        ],
    )(x, batch3d, gate_w, nn_w, nn_b2)
